# baseline (speedup 1.0000x reference)
import functools

import jax
import jax.numpy as jnp
from jax import lax
from jax.experimental import pallas as pl
from jax.experimental.pallas import tpu as pltpu
from jax.experimental.pallas import tpu_sc as plsc

_D = 32
_W = 16
_RS = 17
_NW = 32
_TROWS = 1564
_OFF_BLACK = 768 * _RS
_OFF_SMALL = (1536 * _RS, 1540 * _RS, 1544 * _RS, 1552 * _RS, 1560 * _RS,
              1562 * _RS)
_DBW = 8

_SC_B = 4096
_TB = 1024
_K = 1568


def _sc_kernel(wpi_hbm, bpi_hbm, sm_hbm, table_hbm, out_hbm,
               table_v, wpi_v, bpi_v, sm_v, out_v):
    B = out_hbm.shape[0] // _D
    chunk = B // _NW
    wid = lax.axis_index("s") * 2 + lax.axis_index("c")
    it16 = jnp.arange(16, dtype=jnp.int32)

    pltpu.sync_copy(table_hbm, table_v)

    def do_group(g, carry):
        lanes = g * 16 + it16

        def gather_row(wordv, w0, accs):
            out = []
            for i in range(_DBW):
                word = plsc.load_gather(table_v, [wordv + (w0 + i)])
                lo, hi = plsc.unpack(plsc.bitcast(word, jnp.bfloat16),
                                     format=plsc.PackFormat.INTERLEAVED)
                out.append(accs[2 * i] + lo)
                out.append(accs[2 * i + 1] + hi)
            return tuple(out)

        def make_body(idx_ref, off, w0):
            def body(s, accs):
                idxv = idx_ref[s, pl.ds(g * 16, 16)]
                wordv = idxv * _RS + (s * (12 * _RS) + off)
                return gather_row(wordv, w0, accs)
            return body

        for w0 in range(0, _W, _DBW):
            accs = tuple(jnp.zeros((16,), jnp.float32)
                         for _ in range(2 * _DBW))
            accs = lax.fori_loop(0, 64, make_body(wpi_v, 0, w0), accs,
                                 unroll=4)
            accs = lax.fori_loop(0, 64, make_body(bpi_v, _OFF_BLACK, w0),
                                 accs, unroll=4)
            for s, off in enumerate(_OFF_SMALL):
                idxv = sm_v[s, pl.ds(g * 16, 16)]
                accs = gather_row(idxv * _RS + off, w0, accs)
            for i, acc in enumerate(accs):
                d = 2 * w0 + i
                plsc.store_scatter(out_v, [lanes * _D + d], acc)
        return carry

    base = wid * chunk
    pltpu.sync_copy(wpi_hbm.at[:, pl.ds(base, chunk)], wpi_v)
    pltpu.sync_copy(bpi_hbm.at[:, pl.ds(base, chunk)], bpi_v)
    pltpu.sync_copy(sm_hbm.at[:, pl.ds(base, chunk)], sm_v)
    lax.fori_loop(0, chunk // 16, do_group, 0)
    pltpu.sync_copy(out_v, out_hbm.at[pl.ds(base * _D, chunk * _D)])


def _tc_body(wpi_ref, bpi_ref, sm_ref, w_ref, out_ref, mbuf):
    tb = wpi_ref.shape[0]
    i = pl.program_id(0)
    nt = pl.num_programs(0) - 1

    @pl.when(i < nt)
    def _build():
        cur = (i % 2) * tb
        wpi2 = jnp.concatenate([wpi_ref[...], wpi_ref[...]], axis=1)
        bpi2 = jnp.concatenate([bpi_ref[...], bpi_ref[...]], axis=1)
        hi = jax.lax.broadcasted_iota(jnp.int32, (tb, 128), 1) // 64
        for pp in range(6):
            mbuf[pl.ds(cur, tb), pp * 128:(pp + 1) * 128] = (
                wpi2 == (2 * pp + hi)).astype(jnp.bfloat16)
        for pp in range(6):
            mbuf[pl.ds(cur, tb), 768 + pp * 128:768 + (pp + 1) * 128] = (
                bpi2 == (2 * pp + hi)).astype(jnp.bfloat16)
        c = jax.lax.broadcasted_iota(jnp.int32, (tb, 32), 1)
        segbase = jnp.where(
            c < 4, 0, jnp.where(c < 8, 4, jnp.where(c < 16, 8, jnp.where(
                c < 24, 16, jnp.where(c < 26, 24,
                                      jnp.where(c < 28, 26, 100))))))
        mbuf[pl.ds(cur, tb), 1536:1568] = (
            sm_ref[...] == (c - segbase)).astype(jnp.bfloat16)

    @pl.when(i > 0)
    def _dot():
        prev = ((i - 1) % 2) * tb
        out_ref[...] = jnp.dot(mbuf[pl.ds(prev, tb), :], w_ref[...],
                               preferred_element_type=jnp.float32)


def kernel(white_piece_idx, black_piece_idx, white_castle_idx,
           black_castle_idx, white_ep_idx, black_ep_idx, white_fifty_idx,
           black_fifty_idx, W_white_piece, W_black_piece, W_white_castle,
           W_black_castle, W_white_ep, W_black_ep, W_white_fifty,
           W_black_fifty):
    B = white_piece_idx.shape[0]
    wpi = white_piece_idx.astype(jnp.int32)
    bpi = black_piece_idx.astype(jnp.int32)

    table = jnp.concatenate(
        [W_white_piece.reshape(768, _D), W_black_piece.reshape(768, _D),
         W_white_castle, W_black_castle, W_white_ep, W_black_ep,
         W_white_fifty, W_black_fifty], axis=0)
    tw = jax.lax.bitcast_convert_type(
        table.astype(jnp.bfloat16).reshape(_TROWS, _W, 2), jnp.int32)
    tw = jnp.pad(tw, ((0, 0), (0, _RS - _W))).reshape(-1)

    sm6 = jnp.stack(
        [white_castle_idx, black_castle_idx, white_ep_idx, black_ep_idx,
         white_fifty_idx, black_fifty_idx], axis=0).astype(jnp.int32)

    sc_chunk = _SC_B // _NW
    mesh = plsc.VectorSubcoreMesh(core_axis_name="c", subcore_axis_name="s")
    run_sc = functools.partial(
        pl.kernel, mesh=mesh,
        compiler_params=pltpu.CompilerParams(needs_layout_passes=False),
        out_type=jax.ShapeDtypeStruct((_SC_B * _D,), jnp.float32),
        scratch_types=[
            pltpu.VMEM((_TROWS * _RS,), jnp.int32),
            pltpu.VMEM((64, sc_chunk), jnp.int32),
            pltpu.VMEM((64, sc_chunk), jnp.int32),
            pltpu.VMEM((6, sc_chunk), jnp.int32),
            pltpu.VMEM((sc_chunk * _D,), jnp.float32),
        ],
    )(_sc_kernel)

    Ww = jnp.transpose(W_white_piece, (1, 0, 2)).reshape(768, _D)
    Wb = jnp.transpose(W_black_piece, (1, 0, 2)).reshape(768, _D)
    Wcat = jnp.concatenate(
        [Ww, Wb, W_white_castle, W_black_castle, W_white_ep, W_black_ep,
         W_white_fifty, W_black_fifty,
         jnp.zeros((4, _D), W_white_piece.dtype)],
        axis=0).astype(jnp.bfloat16)

    def rep(x, n):
        return jnp.broadcast_to(x[_SC_B:, None], (B - _SC_B, n))

    sm32 = jnp.concatenate(
        [rep(white_castle_idx, 4), rep(black_castle_idx, 4),
         rep(white_ep_idx, 8), rep(black_ep_idx, 8),
         rep(white_fifty_idx, 2), rep(black_fifty_idx, 2),
         rep(white_fifty_idx, 4)], axis=1).astype(jnp.int32)

    nt = (B - _SC_B) // _TB
    clamp = lambda i: jnp.minimum(i, nt - 1)
    back = lambda i: jnp.maximum(i - 1, 0)
    tc_out = pl.pallas_call(
        _tc_body,
        grid=(nt + 1,),
        in_specs=[
            pl.BlockSpec((_TB, 64), lambda i: (clamp(i), 0)),
            pl.BlockSpec((_TB, 64), lambda i: (clamp(i), 0)),
            pl.BlockSpec((_TB, 32), lambda i: (clamp(i), 0)),
            pl.BlockSpec((_K, _D), lambda i: (0, 0)),
        ],
        out_specs=pl.BlockSpec((_TB, _D), lambda i: (back(i), 0)),
        out_shape=jax.ShapeDtypeStruct((B - _SC_B, _D), jnp.float32),
        scratch_shapes=[pltpu.VMEM((2 * _TB, _K), jnp.bfloat16)],
        compiler_params=pltpu.CompilerParams(
            dimension_semantics=("arbitrary",)),
    )(wpi[_SC_B:], bpi[_SC_B:], sm32, Wcat)

    sc_out = run_sc(wpi[:_SC_B].T, bpi[:_SC_B].T, sm6[:, :_SC_B], tw)
    return jnp.concatenate([sc_out.reshape(_SC_B, _D), tc_out], axis=0)


# speedup vs baseline: 1.0709x; 1.0709x over previous
import functools

import jax
import jax.numpy as jnp
from jax import lax
from jax.experimental import pallas as pl
from jax.experimental.pallas import tpu as pltpu
from jax.experimental.pallas import tpu_sc as plsc

_D = 32
_W = 16
_RS = 17
_NW = 32
_TROWS = 1564
_OFF_BLACK = 768 * _RS
_OFF_SMALL = (1536 * _RS, 1540 * _RS, 1544 * _RS, 1552 * _RS, 1560 * _RS,
              1562 * _RS)
_DBW = 8

_SC_B = 4096
_TB = 1024
_K = 1568


def _sc_kernel(wpi_hbm, bpi_hbm, sm_hbm, table_hbm, out_hbm,
               table_v, wpi_v, bpi_v, sm_v, out_v):
    B = out_hbm.shape[0] // _D
    chunk = B // _NW
    wid = lax.axis_index("s") * 2 + lax.axis_index("c")
    it16 = jnp.arange(16, dtype=jnp.int32)

    pltpu.sync_copy(table_hbm, table_v)

    def do_group(g, carry):
        lanes = g * 16 + it16

        def gather_row(wordv, w0, accs):
            out = []
            for i in range(_DBW):
                word = plsc.load_gather(table_v, [wordv + (w0 + i)])
                lo, hi = plsc.unpack(plsc.bitcast(word, jnp.bfloat16),
                                     format=plsc.PackFormat.INTERLEAVED)
                out.append(accs[2 * i] + lo)
                out.append(accs[2 * i + 1] + hi)
            return tuple(out)

        def make_body(idx_ref, off, w0):
            def body(s, accs):
                idxv = idx_ref[s, pl.ds(g * 16, 16)]
                wordv = idxv * _RS + (s * (12 * _RS) + off)
                return gather_row(wordv, w0, accs)
            return body

        for w0 in range(0, _W, _DBW):
            accs = tuple(jnp.zeros((16,), jnp.float32)
                         for _ in range(2 * _DBW))
            accs = lax.fori_loop(0, 64, make_body(wpi_v, 0, w0), accs,
                                 unroll=4)
            accs = lax.fori_loop(0, 64, make_body(bpi_v, _OFF_BLACK, w0),
                                 accs, unroll=4)
            for s, off in enumerate(_OFF_SMALL):
                idxv = sm_v[s, pl.ds(g * 16, 16)]
                accs = gather_row(idxv * _RS + off, w0, accs)
            for i, acc in enumerate(accs):
                d = 2 * w0 + i
                plsc.store_scatter(out_v, [lanes * _D + d], acc)
        return carry

    base = wid * chunk
    pltpu.sync_copy(wpi_hbm.at[:, pl.ds(base, chunk)], wpi_v)
    pltpu.sync_copy(bpi_hbm.at[:, pl.ds(base, chunk)], bpi_v)
    pltpu.sync_copy(sm_hbm.at[:, pl.ds(base, chunk)], sm_v)
    lax.fori_loop(0, chunk // 16, do_group, 0)
    pltpu.sync_copy(out_v, out_hbm.at[pl.ds(base * _D, chunk * _D)])


def _tc_body(wpi_ref, bpi_ref, sm_ref, w_ref, out_ref):
    tb = wpi_ref.shape[0]
    wpi2 = jnp.concatenate([wpi_ref[...], wpi_ref[...]], axis=1)
    bpi2 = jnp.concatenate([bpi_ref[...], bpi_ref[...]], axis=1)
    hi = jax.lax.broadcasted_iota(jnp.int32, (tb, 128), 1) // 64
    pieces = []
    for pp in range(6):
        pieces.append((wpi2 == (2 * pp + hi)).astype(jnp.bfloat16))
    for pp in range(6):
        pieces.append((bpi2 == (2 * pp + hi)).astype(jnp.bfloat16))
    c = jax.lax.broadcasted_iota(jnp.int32, (tb, 32), 1)
    segbase = jnp.where(
        c < 4, 0, jnp.where(c < 8, 4, jnp.where(c < 16, 8, jnp.where(
            c < 24, 16, jnp.where(c < 26, 24, jnp.where(c < 28, 26, 100))))))
    pieces.append((sm_ref[...] == (c - segbase)).astype(jnp.bfloat16))
    masks = jnp.concatenate(pieces, axis=1)
    out_ref[...] = jnp.dot(masks, w_ref[...],
                           preferred_element_type=jnp.float32)


def kernel(white_piece_idx, black_piece_idx, white_castle_idx,
           black_castle_idx, white_ep_idx, black_ep_idx, white_fifty_idx,
           black_fifty_idx, W_white_piece, W_black_piece, W_white_castle,
           W_black_castle, W_white_ep, W_black_ep, W_white_fifty,
           W_black_fifty):
    B = white_piece_idx.shape[0]
    wpi = white_piece_idx.astype(jnp.int32)
    bpi = black_piece_idx.astype(jnp.int32)

    table = jnp.concatenate(
        [W_white_piece.reshape(768, _D), W_black_piece.reshape(768, _D),
         W_white_castle, W_black_castle, W_white_ep, W_black_ep,
         W_white_fifty, W_black_fifty], axis=0)
    tw = jax.lax.bitcast_convert_type(
        table.astype(jnp.bfloat16).reshape(_TROWS, _W, 2), jnp.int32)
    tw = jnp.pad(tw, ((0, 0), (0, _RS - _W))).reshape(-1)

    sm6 = jnp.stack(
        [white_castle_idx, black_castle_idx, white_ep_idx, black_ep_idx,
         white_fifty_idx, black_fifty_idx], axis=0).astype(jnp.int32)

    sc_chunk = _SC_B // _NW
    mesh = plsc.VectorSubcoreMesh(core_axis_name="c", subcore_axis_name="s")
    run_sc = functools.partial(
        pl.kernel, mesh=mesh,
        compiler_params=pltpu.CompilerParams(needs_layout_passes=False),
        out_type=jax.ShapeDtypeStruct((_SC_B * _D,), jnp.float32),
        scratch_types=[
            pltpu.VMEM((_TROWS * _RS,), jnp.int32),
            pltpu.VMEM((64, sc_chunk), jnp.int32),
            pltpu.VMEM((64, sc_chunk), jnp.int32),
            pltpu.VMEM((6, sc_chunk), jnp.int32),
            pltpu.VMEM((sc_chunk * _D,), jnp.float32),
        ],
    )(_sc_kernel)

    Ww = jnp.transpose(W_white_piece, (1, 0, 2)).reshape(768, _D)
    Wb = jnp.transpose(W_black_piece, (1, 0, 2)).reshape(768, _D)
    Wcat = jnp.concatenate(
        [Ww, Wb, W_white_castle, W_black_castle, W_white_ep, W_black_ep,
         W_white_fifty, W_black_fifty,
         jnp.zeros((4, _D), W_white_piece.dtype)],
        axis=0).astype(jnp.bfloat16)

    def rep(x, n):
        return jnp.broadcast_to(x[_SC_B:, None], (B - _SC_B, n))

    sm32 = jnp.concatenate(
        [rep(white_castle_idx, 4), rep(black_castle_idx, 4),
         rep(white_ep_idx, 8), rep(black_ep_idx, 8),
         rep(white_fifty_idx, 2), rep(black_fifty_idx, 2),
         rep(white_fifty_idx, 4)], axis=1).astype(jnp.int32)

    tc_out = pl.pallas_call(
        _tc_body,
        grid=((B - _SC_B) // _TB,),
        in_specs=[
            pl.BlockSpec((_TB, 64), lambda i: (i, 0)),
            pl.BlockSpec((_TB, 64), lambda i: (i, 0)),
            pl.BlockSpec((_TB, 32), lambda i: (i, 0)),
            pl.BlockSpec((_K, _D), lambda i: (0, 0)),
        ],
        out_specs=pl.BlockSpec((_TB, _D), lambda i: (i, 0)),
        out_shape=jax.ShapeDtypeStruct((B - _SC_B, _D), jnp.float32),
        compiler_params=pltpu.CompilerParams(
            dimension_semantics=("arbitrary",)),
    )(wpi[_SC_B:], bpi[_SC_B:], sm32, Wcat)

    sc_out = run_sc(wpi[:_SC_B].T, bpi[:_SC_B].T, sm6[:, :_SC_B], tw)
    return jnp.concatenate([sc_out.reshape(_SC_B, _D), tc_out], axis=0)
